# Initial kernel scaffold; baseline (speedup 1.0000x reference)
import jax
import jax.numpy as jnp
from jax.experimental import pallas as pl


def kernel(hidden_state, corr_index, nei_index, W_rel, b_rel, W_att, b_att):
    raise NotImplementedError("write your pallas kernel here")


# speedup vs baseline: 20.8098x; 20.8098x over previous
import functools

import jax
import jax.numpy as jnp
from jax.experimental import pallas as pl
from jax.experimental.pallas import tpu as pltpu

P = 1000
M = 64
R = 32
BLK = 200


def _body(alpha_ref, beta_ref, brel_ref, wr_ref, batt_ref,
          x_ref, y_ref, nei_ref, hid_ref, wh_ref, wn_ref, out_ref):
    i = pl.program_id(0)
    x = x_ref[...]
    y = y_ref[...]
    hid = hid_ref[...]

    acc = jnp.zeros((BLK, P), dtype=jnp.float32)
    for r in range(R):
        zr = x * alpha_ref[r] + y * beta_ref[r] + brel_ref[r]
        acc = acc + jnp.maximum(zr, 0.0) * wr_ref[r]

    hrow = hid_ref[pl.ds(i * BLK, BLK), :]
    a = jnp.sum(hrow * wh_ref[...], axis=1, keepdims=True)
    c = jnp.sum(hid * wn_ref[...], axis=1, keepdims=True)
    c_row = c.reshape(1, P)

    z = acc + a + c_row + batt_ref[0]
    mask = nei_ref[...] > 0
    zq = jnp.where(mask & (z != 0.0), z, -1e-6)

    m = jnp.max(zq, axis=1, keepdims=True)
    e = jnp.exp(zq - m)
    d = jnp.sum(e, axis=1, keepdims=True)
    p = jnp.where(mask, e / d, 0.0)

    out_ref[...] = jnp.dot(p, hid, preferred_element_type=jnp.float32)


@jax.jit
def kernel(hidden_state, corr_index, nei_index, W_rel, b_rel, W_att, b_att):
    x = corr_index[:, :, 0]
    y = corr_index[:, :, 1]
    nei = nei_index.astype(jnp.int32)
    alpha = W_rel[:, 0]
    beta = W_rel[:, 1]
    wr = W_att[0, :R]
    wh = W_att[0, R:R + M].reshape(1, M)
    wn = W_att[0, R + M:].reshape(1, M)

    grid = P // BLK
    return pl.pallas_call(
        _body,
        grid=(grid,),
        in_specs=[
            pl.BlockSpec(memory_space=pltpu.SMEM),
            pl.BlockSpec(memory_space=pltpu.SMEM),
            pl.BlockSpec(memory_space=pltpu.SMEM),
            pl.BlockSpec(memory_space=pltpu.SMEM),
            pl.BlockSpec(memory_space=pltpu.SMEM),
            pl.BlockSpec((BLK, P), lambda i: (i, 0)),
            pl.BlockSpec((BLK, P), lambda i: (i, 0)),
            pl.BlockSpec((BLK, P), lambda i: (i, 0)),
            pl.BlockSpec((P, M), lambda i: (0, 0)),
            pl.BlockSpec((1, M), lambda i: (0, 0)),
            pl.BlockSpec((1, M), lambda i: (0, 0)),
        ],
        out_specs=pl.BlockSpec((BLK, M), lambda i: (i, 0)),
        out_shape=jax.ShapeDtypeStruct((P, M), jnp.float32),
        compiler_params=pltpu.CompilerParams(
            dimension_semantics=("arbitrary",),
        ),
    )(alpha, beta, b_rel, wr, b_att, x, y, nei, hidden_state, wh, wn)
